# baseline (speedup 1.0000x reference)
import functools

import jax
import jax.numpy as jnp
from jax import lax
from jax.experimental import pallas as pl
from jax.experimental.pallas import tpu as pltpu
from jax.experimental.pallas import tpu_sc as plsc

B, N, C, G = 8, 4096, 80, 16
IMG = 512.0
H = 64
SUB, LANE = 32, 128

NC, NS, L = 2, 16, 16
BPC = B // NC
QPB = 4
NPQ = N // QPB
NBLK = NPQ // L
POS_R = 1.5 / H


def _tc_body(gtb_ref, gtl_ref, cls_ref, out_ref):
    b = pl.program_id(0)
    shp = (SUB, LANE)

    row = lax.broadcasted_iota(jnp.int32, shp, 0)
    col = lax.broadcasted_iota(jnp.int32, shp, 1)
    lin = row * LANE + col
    cx = ((lin & (H - 1)).astype(jnp.float32) + 0.5) * (1.0 / H)
    cy = (lin >> 6).astype(jnp.float32) * (1.0 / H) + (0.5 / H)

    minv = jnp.full(shp, jnp.inf, dtype=jnp.float32)
    lab = jnp.zeros(shp, dtype=jnp.int32)
    for g in range(G):
        x0 = gtb_ref[0, g, 0] * (1.0 / IMG)
        y0 = gtb_ref[0, g, 1] * (1.0 / IMG)
        x1 = gtb_ref[0, g, 2] * (1.0 / IMG)
        y1 = gtb_ref[0, g, 3] * (1.0 / IMG)
        dx = cx - (x0 + x1) * 0.5
        dy = cy - (y0 + y1) * 0.5
        dist = jnp.sqrt(dx * dx + dy * dy + 1e-12)
        upd = dist < minv
        minv = jnp.where(upd, dist, minv)
        lab = jnp.where(upd, gtl_ref[0, 0, g], lab)

    pos = minv < POS_R
    tgt = jnp.where(pos, lab, 0)
    se = jnp.zeros(shp, dtype=jnp.float32)
    xt = jnp.zeros(shp, dtype=jnp.float32)
    for c in range(C):
        lc = cls_ref[0, c]
        se = se + jnp.exp(lc)
        xt = jnp.where(tgt == c, lc, xt)
    ce = jnp.log(se) - xt
    pt = jnp.exp(-ce)
    om = 1.0 - pt
    fl = 0.25 * om * om * ce
    part = jnp.reshape(jnp.sum(fl) * (1.0 / N), (1, 1))

    @pl.when(b == 0)
    def _():
        out_ref[...] = jnp.zeros_like(out_ref)

    out_ref[...] += part


def _sqrt_nr(x):
    i = plsc.bitcast(x, jnp.int32)
    r = plsc.bitcast(jnp.int32(0x5F3759DF) - (i >> 1), jnp.float32)
    r = r * (1.5 - 0.5 * x * r * r)
    r = r * (1.5 - 0.5 * x * r * r)
    r = r * (1.5 - 0.5 * x * r * r)
    return x * r


def _ln(x):
    i = plsc.bitcast(x, jnp.int32)
    e = (i >> 23) - 127
    m = plsc.bitcast((i & 0x7FFFFF) | 0x3F800000, jnp.float32)
    big = m > 1.41421356
    m = jnp.where(big, m * 0.5, m)
    e = (e + jnp.where(big, 1, 0)).astype(jnp.float32)
    z = m - 1.0
    p = jnp.full((L,), 7.0376836292e-2, dtype=jnp.float32)
    for cc in (-1.1514610310e-1, 1.1676998740e-1, -1.2420140846e-1,
               1.4249322787e-1, -1.6668057665e-1, 2.0000714765e-1,
               -2.4999993993e-1, 3.3333331174e-1):
        p = p * z + jnp.float32(cc)
    z2 = z * z
    r = z2 * z * p - 0.5 * z2
    return (z + r) + e * jnp.float32(-2.12194440e-4) + e * jnp.float32(0.693359375)


def _sc_body(box_hbm, ctr_hbm, gtb_hbm, out_hbm,
             box_v, ctr_v, gtb_v, gcx_v, gcy_v,
             nb0_v, nb1_v, nb2_v, nb3_v, bcx_v, bcy_v,
             stage_v, big_v, ostage_v, shared):
    core = lax.axis_index("c")
    s = lax.axis_index("s")
    bloc = s // QPB
    q = s % QPB
    b = core * BPC + bloc
    pbase = b * N + q * NPQ

    pltpu.sync_copy(box_hbm.at[pl.ds(pbase * 4, NPQ * 4)], box_v)
    pltpu.sync_copy(ctr_hbm.at[pl.ds(pbase, NPQ)], ctr_v)
    pltpu.sync_copy(gtb_hbm.at[pl.ds(b * (G * 4), G * 4)],
                    gtb_v.at[pl.ds(0, G * 4)])

    lane = lax.broadcasted_iota(jnp.int32, (L,), 0)
    inv_img = jnp.float32(1.0 / IMG)
    x0v = plsc.load_gather(gtb_v, [lane * 4]) * inv_img
    y0v = plsc.load_gather(gtb_v, [lane * 4 + 1]) * inv_img
    x1v = plsc.load_gather(gtb_v, [lane * 4 + 2]) * inv_img
    y1v = plsc.load_gather(gtb_v, [lane * 4 + 3]) * inv_img
    nb0_v[pl.ds(0, L)] = x0v
    nb1_v[pl.ds(0, L)] = y0v
    nb2_v[pl.ds(0, L)] = x1v
    nb3_v[pl.ds(0, L)] = y1v
    gcx_v[pl.ds(0, L)] = (x0v + x1v) * 0.5
    gcy_v[pl.ds(0, L)] = (y0v + y1v) * 0.5
    for g in range(G):
        gidx = jnp.full((L,), g, dtype=jnp.int32)
        bcx_v[g, pl.ds(0, L)] = plsc.load_gather(gcx_v, [gidx])
        bcy_v[g, pl.ds(0, L)] = plsc.load_gather(gcy_v, [gidx])

    nbase = q * NPQ
    inv_h = jnp.float32(1.0 / H)
    half_h = jnp.float32(0.5 / H)
    lane4 = lane * 4

    def blk(i, acc):
        cnt_a, bs_a, cs_a = acc
        n = nbase + i * L + lane
        cx = (n & (H - 1)).astype(jnp.float32) * inv_h + half_h
        cy = (n >> 6).astype(jnp.float32) * inv_h + half_h
        minv = jnp.full((L,), 1e9, dtype=jnp.float32)
        best = jnp.zeros((L,), dtype=jnp.int32)
        for g in range(G):
            dx = cx - bcx_v[g, pl.ds(0, L)]
            dy = cy - bcy_v[g, pl.ds(0, L)]
            d2 = dx * dx + dy * dy
            upd = d2 < minv
            minv = jnp.where(upd, d2, minv)
            best = jnp.where(upd, g, best)
        pos = (minv + 1e-12) < (POS_R * POS_R)
        m = pos.astype(jnp.float32)

        b0 = plsc.load_gather(nb0_v, [best])
        b1 = plsc.load_gather(nb1_v, [best])
        b2 = plsc.load_gather(nb2_v, [best])
        b3 = plsc.load_gather(nb3_v, [best])

        bofs = i * (L * 4) + lane4
        sl1 = jnp.zeros((L,), dtype=jnp.float32)
        gb = ((b0 + b2) * 0.5, (b1 + b3) * 0.5, b2 - b0, b3 - b1)
        for ci in range(4):
            bp = plsc.load_gather(box_v, [bofs + ci])
            ad = jnp.abs(bp - gb[ci])
            sl1 = sl1 + jnp.where(ad < 1.0, 0.5 * ad * ad, ad - 0.5)

        l = jnp.maximum(cx - b0, 1e-06)
        r = jnp.maximum(b2 - cx, 1e-06)
        t = jnp.maximum(cy - b1, 1e-06)
        bb = jnp.maximum(b3 - cy, 1e-06)
        ratio = (jnp.minimum(l, r) / jnp.maximum(l, r)) * \
                (jnp.minimum(t, bb) / jnp.maximum(t, bb))
        ct = jnp.clip(_sqrt_nr(ratio), 0.0, 1.0)
        z = plsc.load_gather(ctr_v, [i * L + lane])
        bce = jnp.maximum(z, 0.0) - z * ct + _ln(1.0 + jnp.exp(-jnp.abs(z)))

        return (cnt_a + m, bs_a + sl1 * m, cs_a + bce * m)

    zero = jnp.zeros((L,), dtype=jnp.float32)
    cnt_a, bs_a, cs_a = lax.fori_loop(0, NBLK, blk, (zero, zero, zero))

    cnt_s = jnp.sum(cnt_a)
    bs_s = jnp.sum(bs_a)
    cs_s = jnp.sum(cs_a)
    sel = lane == bloc
    stage_v[0, pl.ds(0, L)] = jnp.where(sel, jnp.full((L,), cnt_s), 0.0)
    stage_v[1, pl.ds(0, L)] = jnp.where(sel, jnp.full((L,), bs_s), 0.0)
    stage_v[2, pl.ds(0, L)] = jnp.where(sel, jnp.full((L,), cs_s), 0.0)
    pltpu.sync_copy(stage_v, shared.at[pl.ds(s * 3, 3)])
    plsc.subcore_barrier()

    @pl.when(s == 0)
    def _():
        pltpu.sync_copy(shared, big_v)
        cnt8 = jnp.zeros((L,), dtype=jnp.float32)
        bs8 = jnp.zeros((L,), dtype=jnp.float32)
        cs8 = jnp.zeros((L,), dtype=jnp.float32)
        for w in range(NS):
            cnt8 = cnt8 + big_v[w * 3 + 0, pl.ds(0, L)]
            bs8 = bs8 + big_v[w * 3 + 1, pl.ds(0, L)]
            cs8 = cs8 + big_v[w * 3 + 2, pl.ds(0, L)]
        has = cnt8 > 0
        bterm = jnp.where(has, bs8 / jnp.maximum(cnt8 * 4.0, 1.0), 0.0)
        cterm = jnp.where(has, cs8 / jnp.maximum(cnt8, 1.0), 0.0)
        tb = jnp.sum(bterm)
        tr = jnp.sum(cterm)
        ov = jnp.where(lane == 0, jnp.full((L,), tb),
                       jnp.where(lane == 1, jnp.full((L,), tr), 0.0))
        ostage_v[pl.ds(0, L)] = ov
        ostage_v[pl.ds(L, L)] = jnp.zeros((L,), jnp.float32)
        for j in range(2, 8):
            ostage_v[pl.ds(j * L, L)] = jnp.zeros((L,), jnp.float32)
        pltpu.sync_copy(ostage_v, out_hbm.at[core])


_sc_call = functools.partial(
    pl.kernel,
    out_type=jax.ShapeDtypeStruct((NC, 128), jnp.float32),
    mesh=plsc.VectorSubcoreMesh(core_axis_name="c", subcore_axis_name="s",
                                num_cores=NC, num_subcores=NS),
    compiler_params=pltpu.CompilerParams(needs_layout_passes=False),
    scratch_types=[
        pltpu.VMEM((NPQ * 4,), jnp.float32),
        pltpu.VMEM((NPQ,), jnp.float32),
        pltpu.VMEM((128,), jnp.float32),
        pltpu.VMEM((128,), jnp.float32),
        pltpu.VMEM((128,), jnp.float32),
        pltpu.VMEM((128,), jnp.float32),
        pltpu.VMEM((128,), jnp.float32),
        pltpu.VMEM((128,), jnp.float32),
        pltpu.VMEM((128,), jnp.float32),
        pltpu.VMEM((G, 128), jnp.float32),
        pltpu.VMEM((G, 128), jnp.float32),
        pltpu.VMEM((3, 128), jnp.float32),
        pltpu.VMEM((NS * 3, 128), jnp.float32),
        pltpu.VMEM((128,), jnp.float32),
        pltpu.VMEM_SHARED((NS * 3, 128), jnp.float32),
    ],
)(_sc_body)


@jax.jit
def kernel(cls_logits, box_preds, centerness, gt_boxes, gt_labels):
    clsT = cls_logits.transpose(0, 2, 1).reshape(B, C, SUB, LANE)
    gtl3 = gt_labels.reshape(B, 1, G)
    sc_out = _sc_call(box_preds.reshape(-1), centerness.reshape(-1),
                      gt_boxes.reshape(-1))
    tc_out = pl.pallas_call(
        _tc_body,
        grid=(B,),
        in_specs=[
            pl.BlockSpec((1, G, 4), lambda b: (b, 0, 0),
                         memory_space=pltpu.SMEM),
            pl.BlockSpec((1, 1, G), lambda b: (b, 0, 0),
                         memory_space=pltpu.SMEM),
            pl.BlockSpec((1, C, SUB, LANE), lambda b: (b, 0, 0, 0)),
        ],
        out_specs=pl.BlockSpec((1, 1), lambda b: (0, 0)),
        out_shape=jax.ShapeDtypeStruct((1, 1), jnp.float32),
    )(gt_boxes, gtl3, clsT)
    tc = tc_out[0, 0]
    tb = sc_out[0, 0] + sc_out[1, 0]
    tr = sc_out[0, 1] + sc_out[1, 1]
    loss = tc / B + 5.0 * tb / B + 1.0 * tr / B
    return (loss, tc / B, tb / B, tr / B)
